# Initial kernel scaffold; baseline (speedup 1.0000x reference)
import jax
import jax.numpy as jnp
from jax.experimental import pallas as pl


def kernel(q, k, v, mask):
    raise NotImplementedError("write your pallas kernel here")


# speedup vs baseline: 11288.1374x; 11288.1374x over previous
import jax
import jax.numpy as jnp
from jax.experimental import pallas as pl

TEMP = 8.0


def _attn_kernel(q_ref, k_ref, v_ref, m_ref, o_ref):
    q = q_ref[0, 0]
    k = k_ref[0, 0]
    v = v_ref[0, 0]
    m = m_ref[0]

    s = jax.lax.dot_general(
        q, k, (((1,), (1,)), ((), ())),
        preferred_element_type=jnp.float32) / TEMP
    neg_inf = jnp.float32(-jnp.inf)
    keep = m > 0.5
    sm = jnp.where(keep, s, neg_inf)
    mx = jnp.max(sm, axis=-1, keepdims=True)
    mx = jnp.where(jnp.isfinite(mx), mx, 0.0)
    ex = jnp.where(keep, jnp.exp(s - mx), 0.0)
    den = jnp.sum(ex, axis=-1, keepdims=True)
    den = jnp.where(den == 0.0, 1.0, den)
    p = ex / den
    o_ref[0, 0] = jax.lax.dot_general(
        p, v, (((1,), (0,)), ((), ())),
        preferred_element_type=jnp.float32)


def kernel(q, k, v, mask):
    B, H, L, d = q.shape
    mf = mask.astype(jnp.float32)
    grid = (B, H)
    out = pl.pallas_call(
        _attn_kernel,
        grid=grid,
        in_specs=[
            pl.BlockSpec((1, 1, L, d), lambda b, h: (b, h, 0, 0)),
            pl.BlockSpec((1, 1, L, d), lambda b, h: (b, h, 0, 0)),
            pl.BlockSpec((1, 1, L, d), lambda b, h: (b, h, 0, 0)),
            pl.BlockSpec((1, L, L), lambda b, h: (b, 0, 0)),
        ],
        out_specs=pl.BlockSpec((1, 1, L, d), lambda b, h: (b, h, 0, 0)),
        out_shape=jax.ShapeDtypeStruct((B, H, L, d), jnp.float32),
    )(q, k, v, mf)
    return out


# speedup vs baseline: 11467.5216x; 1.0159x over previous
import jax
import jax.numpy as jnp
from jax.experimental import pallas as pl

TEMP = 8.0


def _attn_kernel(q_ref, k_ref, v_ref, m_ref, o_ref):
    q = q_ref[0, 0]
    k = k_ref[0, 0]
    v = v_ref[0, 0]
    keep = m_ref[0] != 0

    s = jax.lax.dot_general(
        q, k, (((1,), (1,)), ((), ())),
        preferred_element_type=jnp.float32) * (1.0 / TEMP)
    sm = jnp.where(keep, s, -jnp.inf)
    mx = jnp.max(sm, axis=-1, keepdims=True)
    mx = jnp.where(jnp.isfinite(mx), mx, 0.0)
    ex = jnp.exp(sm - mx)
    den = jnp.sum(ex, axis=-1, keepdims=True)
    r = jnp.where(den == 0.0, 0.0, 1.0 / den)
    p = ex * r
    o_ref[0, 0] = jax.lax.dot_general(
        p, v, (((1,), (0,)), ((), ())),
        preferred_element_type=jnp.float32)


def kernel(q, k, v, mask):
    B, H, L, d = q.shape
    m8 = mask.view(jnp.int8)
    out = pl.pallas_call(
        _attn_kernel,
        grid=(B, H),
        in_specs=[
            pl.BlockSpec((1, 1, L, d), lambda b, h: (b, h, 0, 0)),
            pl.BlockSpec((1, 1, L, d), lambda b, h: (b, h, 0, 0)),
            pl.BlockSpec((1, 1, L, d), lambda b, h: (b, h, 0, 0)),
            pl.BlockSpec((1, L, L), lambda b, h: (b, 0, 0)),
        ],
        out_specs=pl.BlockSpec((1, 1, L, d), lambda b, h: (b, h, 0, 0)),
        out_shape=jax.ShapeDtypeStruct((B, H, L, d), jnp.float32),
    )(q, k, v, m8)
    return out


# speedup vs baseline: 11530.5318x; 1.0055x over previous
import jax
import jax.numpy as jnp
from jax.experimental import pallas as pl

TEMP = 8.0


def _attn_kernel(q_ref, k_ref, v_ref, m_ref, o_ref):
    q = q_ref[0]
    k = k_ref[0]
    v = v_ref[0]
    keep = m_ref[0]

    s = jax.lax.dot_general(
        q, k, (((1,), (1,)), ((), ())),
        preferred_element_type=jnp.float32) * (1.0 / TEMP)
    sm = jnp.where(keep, s, -jnp.inf)
    mx = jnp.max(sm, axis=-1, keepdims=True)
    mx = jnp.where(jnp.isfinite(mx), mx, 0.0)
    ex = jnp.exp(sm - mx)
    den = jnp.sum(ex, axis=-1, keepdims=True)
    r = jnp.where(den == 0.0, 0.0, 1.0 / den)
    p = ex * r
    o_ref[0] = jax.lax.dot_general(
        p, v, (((1,), (0,)), ((), ())),
        preferred_element_type=jnp.float32)


def kernel(q, k, v, mask):
    B, H, L, d = q.shape
    q3 = q.reshape(B * H, L, d)
    k3 = k.reshape(B * H, L, d)
    v3 = v.reshape(B * H, L, d)
    out = pl.pallas_call(
        _attn_kernel,
        grid=(B * H,),
        in_specs=[
            pl.BlockSpec((1, L, d), lambda i: (i, 0, 0)),
            pl.BlockSpec((1, L, d), lambda i: (i, 0, 0)),
            pl.BlockSpec((1, L, d), lambda i: (i, 0, 0)),
            pl.BlockSpec((1, L, L), lambda i: (i // H, 0, 0)),
        ],
        out_specs=pl.BlockSpec((1, L, d), lambda i: (i, 0, 0)),
        out_shape=jax.ShapeDtypeStruct((B * H, L, d), jnp.float32),
    )(q3, k3, v3, mask)
    return out.reshape(B, H, L, d)
